# baseline (speedup 1.0000x reference)
import jax
import jax.numpy as jnp
from jax.experimental import pallas as pl
from jax.experimental.pallas import tpu as pltpu

T = 4096
D_MODEL = 2048
D_FF = 2048
D_OUT = 2048
E = 8
GT = 1024
TT = 1024


def _gate_kernel(x_ref, gw_ref, gb_ref, gate_ref, xb_ref):
    x32 = x_ref[...]
    logits = jnp.dot(x32, gw_ref[...],
                     preferred_element_type=jnp.float32) + gb_ref[0]
    m = jnp.max(logits, axis=-1, keepdims=True)
    p = jnp.exp(logits - m)
    gate_ref[...] = p / jnp.sum(p, axis=-1, keepdims=True)
    xb_ref[...] = x32.astype(jnp.bfloat16)


def _moe_kernel(xb_ref, g_ref, w1_ref, b1_ref, w2_ref, b2_ref, out_ref):
    e = pl.program_id(1)

    xb = xb_ref[...]
    h = jnp.dot(xb, w1_ref[0], preferred_element_type=jnp.float32)
    h = jnp.maximum(h + b1_ref[0], 0.0).astype(jnp.bfloat16)
    y = jnp.dot(h, w2_ref[0], preferred_element_type=jnp.float32) + b2_ref[0]

    g = g_ref[...]
    lane = jax.lax.broadcasted_iota(jnp.int32, g.shape, 1)
    ge = jnp.sum(jnp.where(lane == e, g, 0.0), axis=-1, keepdims=True)
    contrib = ge * y

    @pl.when(e == 0)
    def _():
        out_ref[...] = contrib

    @pl.when(e != 0)
    def _():
        out_ref[...] += contrib


def _cast_kernel(w1_ref, w2_ref, o1_ref, o2_ref):
    o1_ref[...] = w1_ref[...].astype(jnp.bfloat16)
    o2_ref[...] = w2_ref[...].astype(jnp.bfloat16)


def kernel(x, gate_W, gate_b, W1, b1, W2, b2):
    NCH = 2 * E
    w1, w2 = pl.pallas_call(
        _cast_kernel,
        grid=(NCH,),
        in_specs=[
            pl.BlockSpec((1, D_MODEL * E // NCH, D_FF), lambda c: (0, c, 0)),
            pl.BlockSpec((1, D_FF * E // NCH, D_OUT), lambda c: (0, c, 0)),
        ],
        out_specs=[
            pl.BlockSpec((1, D_MODEL * E // NCH, D_FF), lambda c: (0, c, 0)),
            pl.BlockSpec((1, D_FF * E // NCH, D_OUT), lambda c: (0, c, 0)),
        ],
        out_shape=[
            jax.ShapeDtypeStruct((1, E * D_MODEL, D_FF), jnp.bfloat16),
            jax.ShapeDtypeStruct((1, E * D_FF, D_OUT), jnp.bfloat16),
        ],
    )(W1.reshape(1, E * D_MODEL, D_FF), W2.reshape(1, E * D_FF, D_OUT))
    w1 = w1.reshape(E, D_MODEL, D_FF)
    w2 = w2.reshape(E, D_FF, D_OUT)
    b1r = b1.reshape(E, 1, D_FF)
    b2r = b2.reshape(E, 1, D_OUT)
    gbr = gate_b.reshape(1, E)

    gate, xb = pl.pallas_call(
        _gate_kernel,
        grid=(T // GT,),
        in_specs=[
            pl.BlockSpec((GT, D_MODEL), lambda t: (t, 0)),
            pl.BlockSpec((D_MODEL, E), lambda t: (0, 0)),
            pl.BlockSpec((1, E), lambda t: (0, 0)),
        ],
        out_specs=[
            pl.BlockSpec((GT, E), lambda t: (t, 0)),
            pl.BlockSpec((GT, D_MODEL), lambda t: (t, 0)),
        ],
        out_shape=[
            jax.ShapeDtypeStruct((T, E), jnp.float32),
            jax.ShapeDtypeStruct((T, D_MODEL), jnp.bfloat16),
        ],
        compiler_params=pltpu.CompilerParams(
            dimension_semantics=("arbitrary",)),
    )(x, gate_W, gbr)

    return pl.pallas_call(
        _moe_kernel,
        grid=(T // TT, E),
        in_specs=[
            pl.BlockSpec((TT, D_MODEL), lambda t, e: (t, 0),
                         pipeline_mode=pl.Buffered(buffer_count=1)),
            pl.BlockSpec((TT, E), lambda t, e: (t, 0)),
            pl.BlockSpec((1, D_MODEL, D_FF), lambda t, e: (e, 0, 0)),
            pl.BlockSpec((1, 1, D_FF), lambda t, e: (e, 0, 0)),
            pl.BlockSpec((1, D_FF, D_OUT), lambda t, e: (e, 0, 0)),
            pl.BlockSpec((1, 1, D_OUT), lambda t, e: (e, 0, 0)),
        ],
        out_specs=pl.BlockSpec((TT, D_OUT), lambda t, e: (t, 0),
                               pipeline_mode=pl.Buffered(buffer_count=1)),
        out_shape=jax.ShapeDtypeStruct((T, D_OUT), jnp.float32),
        compiler_params=pltpu.CompilerParams(
            dimension_semantics=("parallel", "arbitrary")),
    )(xb, gate, w1, b1r, w2, b2r)


# speedup vs baseline: 1.0147x; 1.0147x over previous
import jax
import jax.numpy as jnp
from jax.experimental import pallas as pl
from jax.experimental.pallas import tpu as pltpu

T = 4096
D_MODEL = 2048
D_FF = 2048
D_OUT = 2048
E = 8
GT = 1024
TT = 512


def _gate_kernel(x_ref, gw_ref, gb_ref, gate_ref, xb_ref):
    x32 = x_ref[...]
    logits = jnp.dot(x32, gw_ref[...],
                     preferred_element_type=jnp.float32) + gb_ref[0]
    m = jnp.max(logits, axis=-1, keepdims=True)
    p = jnp.exp(logits - m)
    gate_ref[...] = p / jnp.sum(p, axis=-1, keepdims=True)
    xb_ref[...] = x32.astype(jnp.bfloat16)


def _moe_kernel(xb_ref, g_ref, w1_ref, b1_ref, w2_ref, b2_ref, out_ref):
    e = pl.program_id(1)

    xb = xb_ref[...]
    h = jnp.dot(xb, w1_ref[0], preferred_element_type=jnp.float32)
    h = jnp.maximum(h + b1_ref[0], 0.0).astype(jnp.bfloat16)
    y = jnp.dot(h, w2_ref[0], preferred_element_type=jnp.float32) + b2_ref[0]

    g = g_ref[...]
    lane = jax.lax.broadcasted_iota(jnp.int32, g.shape, 1)
    ge = jnp.sum(jnp.where(lane == e, g, 0.0), axis=-1, keepdims=True)
    contrib = ge * y

    @pl.when(e == 0)
    def _():
        out_ref[...] = contrib

    @pl.when(e != 0)
    def _():
        out_ref[...] += contrib


def _cast_kernel(w1_ref, w2_ref, o1_ref, o2_ref):
    o1_ref[...] = w1_ref[...].astype(jnp.bfloat16)
    o2_ref[...] = w2_ref[...].astype(jnp.bfloat16)


def kernel(x, gate_W, gate_b, W1, b1, W2, b2):
    NCH = 2 * E
    w1, w2 = pl.pallas_call(
        _cast_kernel,
        grid=(NCH,),
        in_specs=[
            pl.BlockSpec((1, D_MODEL * E // NCH, D_FF), lambda c: (0, c, 0)),
            pl.BlockSpec((1, D_FF * E // NCH, D_OUT), lambda c: (0, c, 0)),
        ],
        out_specs=[
            pl.BlockSpec((1, D_MODEL * E // NCH, D_FF), lambda c: (0, c, 0)),
            pl.BlockSpec((1, D_FF * E // NCH, D_OUT), lambda c: (0, c, 0)),
        ],
        out_shape=[
            jax.ShapeDtypeStruct((1, E * D_MODEL, D_FF), jnp.bfloat16),
            jax.ShapeDtypeStruct((1, E * D_FF, D_OUT), jnp.bfloat16),
        ],
    )(W1.reshape(1, E * D_MODEL, D_FF), W2.reshape(1, E * D_FF, D_OUT))
    w1 = w1.reshape(E, D_MODEL, D_FF)
    w2 = w2.reshape(E, D_FF, D_OUT)
    b1r = b1.reshape(E, 1, D_FF)
    b2r = b2.reshape(E, 1, D_OUT)
    gbr = gate_b.reshape(1, E)

    gate, xb = pl.pallas_call(
        _gate_kernel,
        grid=(T // GT,),
        in_specs=[
            pl.BlockSpec((GT, D_MODEL), lambda t: (t, 0)),
            pl.BlockSpec((D_MODEL, E), lambda t: (0, 0)),
            pl.BlockSpec((1, E), lambda t: (0, 0)),
        ],
        out_specs=[
            pl.BlockSpec((GT, E), lambda t: (t, 0)),
            pl.BlockSpec((GT, D_MODEL), lambda t: (t, 0)),
        ],
        out_shape=[
            jax.ShapeDtypeStruct((T, E), jnp.float32),
            jax.ShapeDtypeStruct((T, D_MODEL), jnp.bfloat16),
        ],
        compiler_params=pltpu.CompilerParams(
            dimension_semantics=("arbitrary",)),
    )(x, gate_W, gbr)

    return pl.pallas_call(
        _moe_kernel,
        grid=(T // TT, E),
        in_specs=[
            pl.BlockSpec((TT, D_MODEL), lambda t, e: (t, 0)),
            pl.BlockSpec((TT, E), lambda t, e: (t, 0)),
            pl.BlockSpec((1, D_MODEL, D_FF), lambda t, e: (e, 0, 0)),
            pl.BlockSpec((1, 1, D_FF), lambda t, e: (e, 0, 0)),
            pl.BlockSpec((1, D_FF, D_OUT), lambda t, e: (e, 0, 0)),
            pl.BlockSpec((1, 1, D_OUT), lambda t, e: (e, 0, 0)),
        ],
        out_specs=pl.BlockSpec((TT, D_OUT), lambda t, e: (t, 0)),
        out_shape=jax.ShapeDtypeStruct((T, D_OUT), jnp.float32),
        compiler_params=pltpu.CompilerParams(
            dimension_semantics=("arbitrary", "arbitrary")),
    )(xb, gate, w1, b1r, w2, b2r)
